# baseline (speedup 1.0000x reference)
import functools

import jax
import jax.numpy as jnp
from jax import lax
from jax.experimental import pallas as pl
from jax.experimental.pallas import tpu as pltpu
from jax.experimental.pallas import tpu_sc as plsc

B, V, NF, NO, PDIM, ZDIM = 16, 778, 1538, 2048, 61, 64
KL_COEF = 0.005
BIG_I = 2 ** 30
NFP = 1600
QC = 512
VP = 784
VP3 = VP * 3
NF3P = NFP * 3

_DNT = (((0,), (0,)), ((), ()))


def _sc_normals_body(vab_hbm, faces_hbm, out_hbm, verts_v, faces_v, vn_v):
    wid = lax.axis_index("s") * 2 + lax.axis_index("c")
    b = wid // 2
    m = wid % 2
    pltpu.sync_copy(vab_hbm.at[b, m], verts_v)
    pltpu.sync_copy(faces_hbm.at[b], faces_v)

    zero16 = jnp.zeros((16,), jnp.float32)

    def _zero(i, c):
        vn_v[pl.ds(i * 16, 16)] = zero16
        return c

    lax.fori_loop(0, VP3 // 16, _zero, 0)

    def _face_chunk(i, c):
        base = i * 16
        i0 = faces_v[pl.ds(base, 16)] * 3
        i1 = faces_v[pl.ds(NFP + base, 16)] * 3
        i2 = faces_v[pl.ds(2 * NFP + base, 16)] * 3
        v0x = plsc.load_gather(verts_v, [i0])
        v0y = plsc.load_gather(verts_v, [i0 + 1])
        v0z = plsc.load_gather(verts_v, [i0 + 2])
        v1x = plsc.load_gather(verts_v, [i1])
        v1y = plsc.load_gather(verts_v, [i1 + 1])
        v1z = plsc.load_gather(verts_v, [i1 + 2])
        v2x = plsc.load_gather(verts_v, [i2])
        v2y = plsc.load_gather(verts_v, [i2 + 1])
        v2z = plsc.load_gather(verts_v, [i2 + 2])
        e1x, e1y, e1z = v1x - v0x, v1y - v0y, v1z - v0z
        e2x, e2y, e2z = v2x - v0x, v2y - v0y, v2z - v0z
        fx = e1y * e2z - e1z * e2y
        fy = e1z * e2x - e1x * e2z
        fz = e1x * e2y - e1y * e2x
        for ic in (i0, i1, i2):
            plsc.addupdate_scatter(vn_v, [ic], fx)
            plsc.addupdate_scatter(vn_v, [ic + 1], fy)
            plsc.addupdate_scatter(vn_v, [ic + 2], fz)
        return c

    lax.fori_loop(0, NFP // 16, _face_chunk, 0)
    pltpu.sync_copy(vn_v, out_hbm.at[b, m])


def _sc_normals(vab_flat, faces_flat):
    mesh = plsc.VectorSubcoreMesh(core_axis_name="c", subcore_axis_name="s")
    fn = functools.partial(
        pl.kernel,
        mesh=mesh,
        compiler_params=pltpu.CompilerParams(needs_layout_passes=False),
        out_type=jax.ShapeDtypeStruct((B, 2, VP3), jnp.float32),
        scratch_types=[
            pltpu.VMEM((VP3,), jnp.float32),
            pltpu.VMEM((NF3P,), jnp.int32),
            pltpu.VMEM((VP3,), jnp.float32),
        ],
    )(_sc_normals_body)
    return fn(vab_flat, faces_flat)


def _loss_kernel(va_ref, vb_ref, vn_ref, objt_ref, objr_ref, vw_ref,
                 rx_ref, xx_ref, mu_ref, lv_ref,
                 loss_ref, param_ref, ho_ref, recon_ref, kld_ref):
    b = pl.program_id(0)

    @pl.when(b == 0)
    def _init():
        z = jnp.zeros((1, 1), jnp.float32)
        loss_ref[:, :] = z
        param_ref[:, :] = z
        ho_ref[:, :] = z
        recon_ref[:, :] = z
        kld_ref[:, :] = z

    va = va_ref[0]
    vb = vb_ref[0]
    vw = vw_ref[:]
    rx = rx_ref[0]
    xx = xx_ref[0]
    mu = mu_ref[0]
    lv = lv_ref[0]

    def _unit(vn):
        n = jnp.sqrt(jnp.sum(vn * vn, axis=1, keepdims=True))
        return vn / jnp.maximum(n, 1e-6)

    wa = jnp.concatenate([va, _unit(vn_ref[0, 0])], axis=1)
    wb = jnp.concatenate([vb, _unit(vn_ref[0, 1])], axis=1)

    h2a = jnp.sum(va * va, axis=1, keepdims=True)
    h2b = jnp.sum(vb * vb, axis=1, keepdims=True)
    iota_p = jax.lax.broadcasted_iota(jnp.int32, (V, QC), 0)
    iota_q = jax.lax.broadcasted_iota(jnp.int32, (V, QC), 1)

    st_a = [jnp.full((V, 1), 1e30, jnp.float32), jnp.zeros((V, 3), jnp.float32)]
    st_b = [jnp.full((V, 1), 1e30, jnp.float32), jnp.zeros((V, 3), jnp.float32)]
    ldo = 0.0
    for k in range(NO // QC):
        objc = objt_ref[0, :, k * QC:(k + 1) * QC]
        objr = objr_ref[0, k * QC:(k + 1) * QC, :]
        o2 = jnp.sum(objc * objc, axis=0, keepdims=True)

        def _signed(verts, h2, w6, st):
            d = jnp.maximum(h2 + o2 - 2.0 * jnp.dot(verts, objc), 0.0)
            cidx = jnp.argmin(d, axis=0)[None, :]
            cmask = (iota_p == cidx).astype(jnp.float32)
            sel = jax.lax.dot_general(cmask, w6, _DNT)
            dx = objr[:, 0:1] - sel[:, 0:1]
            dy = objr[:, 1:2] - sel[:, 1:2]
            dz = objr[:, 2:3] - sel[:, 2:3]
            mag = jnp.sqrt(dx * dx + dy * dy + dz * dz)
            dotn = sel[:, 3:4] * dx + sel[:, 4:5] * dy + sel[:, 5:6] * dz
            sgn = jnp.where(dotn > 0.0, 1.0,
                            jnp.where(dotn < 0.0, -1.0, 0.0))
            rmin = jnp.min(d, axis=1, keepdims=True)
            ridx = jnp.argmin(d, axis=1)[:, None]
            rmask = (iota_q == ridx).astype(jnp.float32)
            rsel = jnp.dot(rmask, objr)
            upd = rmin < st[0]
            st[0] = jnp.where(upd, rmin, st[0])
            st[1] = jnp.where(upd, rsel, st[1])
            return mag * sgn

        o2h_a = _signed(va, h2a, wa, st_a)
        o2h_b = _signed(vb, h2b, wb, st_b)

        w_dist = (o2h_b < 0.01) & (o2h_b > -0.005)
        w = jnp.where(w_dist, 1.0, 0.1)
        w = jnp.where(o2h_a < 0.0, 1.5, w)
        ldo = ldo + jnp.sum(jnp.abs(o2h_a - o2h_b) * w)

    def _rownorm(verts, st):
        e = verts - st[1]
        return jnp.sqrt(jnp.sum(e * e, axis=1, keepdims=True))

    h2o_a = _rownorm(va, st_a)
    h2o_b = _rownorm(vb, st_b)
    w2 = jnp.exp(0.4 * jnp.log(vw))
    ldh = jnp.sum(jnp.abs(jnp.abs(h2o_a) - jnp.abs(h2o_b)) * w2)

    scale = 1.0 - KL_COEF
    ho_p = (35.0 * scale / (B * V)) * ldh + (30.0 * scale / (B * NO)) * ldo

    dpx = rx - xx
    param_p = jnp.sum(dpx * dpx) / B
    dv = va - vb
    recon_p = jnp.sum(dv * dv) / B
    kld_p = -0.5 * jnp.sum(1.0 + lv - mu * mu - jnp.exp(lv)) / B

    def _acc(ref, val):
        ref[:, :] = ref[:, :] + jnp.full((1, 1), 1.0, jnp.float32) * val

    _acc(loss_ref, (recon_p + kld_p) + 0.1 * param_p + 10.0 * ho_p)
    _acc(param_ref, param_p)
    _acc(ho_ref, ho_p)
    _acc(recon_ref, recon_p)
    _acc(kld_ref, kld_p)


def kernel(recon_x, x, mu, logvar, recon_xyz, hand_xyz, hand_faces, obj_pts,
           v_weights):
    vab = jnp.stack([recon_xyz, hand_xyz], axis=1)
    vab_flat = jnp.pad(vab, ((0, 0), (0, 0), (0, VP - V), (0, 0))
                       ).reshape(B, 2, VP3)
    faces_pad = jnp.pad(hand_faces, ((0, 0), (0, NFP - NF), (0, 0)),
                        constant_values=V)
    faces_flat = jnp.swapaxes(faces_pad, 1, 2).reshape(B, NF3P)
    vn = _sc_normals(vab_flat, faces_flat)
    vn = vn.reshape(B, 2, VP, 3)[:, :, :V, :]

    obj_t = jnp.swapaxes(obj_pts, 1, 2)
    vw_col = v_weights.reshape(V, 1)
    rx3 = recon_x.reshape(B, 1, PDIM)
    x3 = x.reshape(B, 1, PDIM)
    mu3 = mu.reshape(B, 1, ZDIM)
    lv3 = logvar.reshape(B, 1, ZDIM)

    out_shape = [jax.ShapeDtypeStruct((1, 1), jnp.float32)] * 5
    scal = pl.BlockSpec((1, 1), lambda b: (0, 0))
    outs = pl.pallas_call(
        _loss_kernel,
        grid=(B,),
        in_specs=[
            pl.BlockSpec((1, V, 3), lambda b: (b, 0, 0)),
            pl.BlockSpec((1, V, 3), lambda b: (b, 0, 0)),
            pl.BlockSpec((1, 2, V, 3), lambda b: (b, 0, 0, 0)),
            pl.BlockSpec((1, 3, NO), lambda b: (b, 0, 0)),
            pl.BlockSpec((1, NO, 3), lambda b: (b, 0, 0)),
            pl.BlockSpec((V, 1), lambda b: (0, 0)),
            pl.BlockSpec((1, 1, PDIM), lambda b: (b, 0, 0)),
            pl.BlockSpec((1, 1, PDIM), lambda b: (b, 0, 0)),
            pl.BlockSpec((1, 1, ZDIM), lambda b: (b, 0, 0)),
            pl.BlockSpec((1, 1, ZDIM), lambda b: (b, 0, 0)),
        ],
        out_specs=[scal] * 5,
        out_shape=out_shape,
    )(recon_xyz, hand_xyz, vn, obj_t, obj_pts, vw_col,
      rx3, x3, mu3, lv3)

    loss, param_loss, ho_loss, recon_loss, kld = [o.reshape(()) for o in outs]
    return (loss, param_loss, ho_loss, recon_loss, kld)


# speedup vs baseline: 1.4645x; 1.4645x over previous
import functools

import jax
import jax.numpy as jnp
from jax import lax
from jax.experimental import pallas as pl
from jax.experimental.pallas import tpu as pltpu
from jax.experimental.pallas import tpu_sc as plsc

B, V, NF, NO, PDIM, ZDIM = 16, 778, 1538, 2048, 61, 64
KL_COEF = 0.005
BIG_I = 2 ** 30
NFP = 1600
QC = 1024
VP = 784
VP3 = VP * 3
NF3P = NFP * 3

_DNT = (((0,), (0,)), ((), ()))


def _sc_normals_body(vab_hbm, faces_hbm, out_hbm, verts_v, faces_v, vn_v):
    wid = lax.axis_index("s") * 2 + lax.axis_index("c")
    b = wid // 2
    m = wid % 2
    pltpu.sync_copy(vab_hbm.at[b, m], verts_v)
    pltpu.sync_copy(faces_hbm.at[b], faces_v)

    zero16 = jnp.zeros((16,), jnp.float32)

    def _zero(i, c):
        vn_v[pl.ds(i * 16, 16)] = zero16
        return c

    lax.fori_loop(0, VP3 // 16, _zero, 0)

    def _face_chunk(i, c):
        base = i * 16
        i0 = faces_v[pl.ds(base, 16)] * 3
        i1 = faces_v[pl.ds(NFP + base, 16)] * 3
        i2 = faces_v[pl.ds(2 * NFP + base, 16)] * 3
        v0x = plsc.load_gather(verts_v, [i0])
        v0y = plsc.load_gather(verts_v, [i0 + 1])
        v0z = plsc.load_gather(verts_v, [i0 + 2])
        v1x = plsc.load_gather(verts_v, [i1])
        v1y = plsc.load_gather(verts_v, [i1 + 1])
        v1z = plsc.load_gather(verts_v, [i1 + 2])
        v2x = plsc.load_gather(verts_v, [i2])
        v2y = plsc.load_gather(verts_v, [i2 + 1])
        v2z = plsc.load_gather(verts_v, [i2 + 2])
        e1x, e1y, e1z = v1x - v0x, v1y - v0y, v1z - v0z
        e2x, e2y, e2z = v2x - v0x, v2y - v0y, v2z - v0z
        fx = e1y * e2z - e1z * e2y
        fy = e1z * e2x - e1x * e2z
        fz = e1x * e2y - e1y * e2x
        for ic in (i0, i1, i2):
            plsc.addupdate_scatter(vn_v, [ic], fx)
            plsc.addupdate_scatter(vn_v, [ic + 1], fy)
            plsc.addupdate_scatter(vn_v, [ic + 2], fz)
        return c

    lax.fori_loop(0, NFP // 16, _face_chunk, 0)
    pltpu.sync_copy(vn_v, out_hbm.at[b, m])


def _sc_normals(vab_flat, faces_flat):
    mesh = plsc.VectorSubcoreMesh(core_axis_name="c", subcore_axis_name="s")
    fn = functools.partial(
        pl.kernel,
        mesh=mesh,
        compiler_params=pltpu.CompilerParams(needs_layout_passes=False),
        out_type=jax.ShapeDtypeStruct((B, 2, VP3), jnp.float32),
        scratch_types=[
            pltpu.VMEM((VP3,), jnp.float32),
            pltpu.VMEM((NF3P,), jnp.int32),
            pltpu.VMEM((VP3,), jnp.float32),
        ],
    )(_sc_normals_body)
    return fn(vab_flat, faces_flat)


def _loss_kernel(va_ref, vb_ref, vn_ref, objt_ref, objr_ref, vw_ref,
                 rx_ref, xx_ref, mu_ref, lv_ref,
                 loss_ref, param_ref, ho_ref, recon_ref, kld_ref):
    b = pl.program_id(0)

    @pl.when(b == 0)
    def _init():
        z = jnp.zeros((1, 1), jnp.float32)
        loss_ref[:, :] = z
        param_ref[:, :] = z
        ho_ref[:, :] = z
        recon_ref[:, :] = z
        kld_ref[:, :] = z

    va = va_ref[0]
    vb = vb_ref[0]
    vw = vw_ref[:]
    rx = rx_ref[0]
    xx = xx_ref[0]
    mu = mu_ref[0]
    lv = lv_ref[0]

    def _unit(vn):
        n = jnp.sqrt(jnp.sum(vn * vn, axis=1, keepdims=True))
        return vn / jnp.maximum(n, 1e-6)

    wa = jnp.concatenate([va, _unit(vn_ref[0, 0])], axis=1)
    wb = jnp.concatenate([vb, _unit(vn_ref[0, 1])], axis=1)

    h2a = jnp.sum(va * va, axis=1, keepdims=True)
    h2b = jnp.sum(vb * vb, axis=1, keepdims=True)
    iota_p = jax.lax.broadcasted_iota(jnp.int32, (V, QC), 0)
    iota_q = jax.lax.broadcasted_iota(jnp.int32, (V, QC), 1)

    st_a = [jnp.full((V, 1), 1e30, jnp.float32), jnp.zeros((V, 3), jnp.float32)]
    st_b = [jnp.full((V, 1), 1e30, jnp.float32), jnp.zeros((V, 3), jnp.float32)]
    ldo = 0.0
    for k in range(NO // QC):
        objc = objt_ref[0, :, k * QC:(k + 1) * QC]
        objr = objr_ref[0, k * QC:(k + 1) * QC, :]
        o2 = jnp.sum(objc * objc, axis=0, keepdims=True)

        def _signed(verts, h2, w6, st):
            d = jnp.maximum(h2 + o2 - 2.0 * jnp.dot(verts, objc), 0.0)
            cmin = jnp.min(d, axis=0, keepdims=True)
            cidx = jnp.min(jnp.where(d == cmin, iota_p, BIG_I),
                           axis=0, keepdims=True)
            cmask = (iota_p == cidx).astype(jnp.float32)
            sel = jax.lax.dot_general(cmask, w6, _DNT)
            dx = objr[:, 0:1] - sel[:, 0:1]
            dy = objr[:, 1:2] - sel[:, 1:2]
            dz = objr[:, 2:3] - sel[:, 2:3]
            mag = jnp.sqrt(dx * dx + dy * dy + dz * dz)
            dotn = sel[:, 3:4] * dx + sel[:, 4:5] * dy + sel[:, 5:6] * dz
            sgn = jnp.where(dotn > 0.0, 1.0,
                            jnp.where(dotn < 0.0, -1.0, 0.0))
            rmin = jnp.min(d, axis=1, keepdims=True)
            ridx = jnp.min(jnp.where(d == rmin, iota_q, BIG_I),
                           axis=1, keepdims=True)
            rmask = (iota_q == ridx).astype(jnp.float32)
            rsel = jnp.dot(rmask, objr)
            upd = rmin < st[0]
            st[0] = jnp.where(upd, rmin, st[0])
            st[1] = jnp.where(upd, rsel, st[1])
            return mag * sgn

        o2h_a = _signed(va, h2a, wa, st_a)
        o2h_b = _signed(vb, h2b, wb, st_b)

        w_dist = (o2h_b < 0.01) & (o2h_b > -0.005)
        w = jnp.where(w_dist, 1.0, 0.1)
        w = jnp.where(o2h_a < 0.0, 1.5, w)
        ldo = ldo + jnp.sum(jnp.abs(o2h_a - o2h_b) * w)

    def _rownorm(verts, st):
        e = verts - st[1]
        return jnp.sqrt(jnp.sum(e * e, axis=1, keepdims=True))

    h2o_a = _rownorm(va, st_a)
    h2o_b = _rownorm(vb, st_b)
    w2 = jnp.exp(0.4 * jnp.log(vw))
    ldh = jnp.sum(jnp.abs(jnp.abs(h2o_a) - jnp.abs(h2o_b)) * w2)

    scale = 1.0 - KL_COEF
    ho_p = (35.0 * scale / (B * V)) * ldh + (30.0 * scale / (B * NO)) * ldo

    dpx = rx - xx
    param_p = jnp.sum(dpx * dpx) / B
    dv = va - vb
    recon_p = jnp.sum(dv * dv) / B
    kld_p = -0.5 * jnp.sum(1.0 + lv - mu * mu - jnp.exp(lv)) / B

    def _acc(ref, val):
        ref[:, :] = ref[:, :] + jnp.full((1, 1), 1.0, jnp.float32) * val

    _acc(loss_ref, (recon_p + kld_p) + 0.1 * param_p + 10.0 * ho_p)
    _acc(param_ref, param_p)
    _acc(ho_ref, ho_p)
    _acc(recon_ref, recon_p)
    _acc(kld_ref, kld_p)


def kernel(recon_x, x, mu, logvar, recon_xyz, hand_xyz, hand_faces, obj_pts,
           v_weights):
    vab = jnp.stack([recon_xyz, hand_xyz], axis=1)
    vab_flat = jnp.pad(vab, ((0, 0), (0, 0), (0, VP - V), (0, 0))
                       ).reshape(B, 2, VP3)
    faces_pad = jnp.pad(hand_faces, ((0, 0), (0, NFP - NF), (0, 0)),
                        constant_values=V)
    faces_flat = jnp.swapaxes(faces_pad, 1, 2).reshape(B, NF3P)
    vn = _sc_normals(vab_flat, faces_flat)
    vn = vn.reshape(B, 2, VP, 3)[:, :, :V, :]

    obj_t = jnp.swapaxes(obj_pts, 1, 2)
    vw_col = v_weights.reshape(V, 1)
    rx3 = recon_x.reshape(B, 1, PDIM)
    x3 = x.reshape(B, 1, PDIM)
    mu3 = mu.reshape(B, 1, ZDIM)
    lv3 = logvar.reshape(B, 1, ZDIM)

    out_shape = [jax.ShapeDtypeStruct((1, 1), jnp.float32)] * 5
    scal = pl.BlockSpec((1, 1), lambda b: (0, 0))
    outs = pl.pallas_call(
        _loss_kernel,
        grid=(B,),
        in_specs=[
            pl.BlockSpec((1, V, 3), lambda b: (b, 0, 0)),
            pl.BlockSpec((1, V, 3), lambda b: (b, 0, 0)),
            pl.BlockSpec((1, 2, V, 3), lambda b: (b, 0, 0, 0)),
            pl.BlockSpec((1, 3, NO), lambda b: (b, 0, 0)),
            pl.BlockSpec((1, NO, 3), lambda b: (b, 0, 0)),
            pl.BlockSpec((V, 1), lambda b: (0, 0)),
            pl.BlockSpec((1, 1, PDIM), lambda b: (b, 0, 0)),
            pl.BlockSpec((1, 1, PDIM), lambda b: (b, 0, 0)),
            pl.BlockSpec((1, 1, ZDIM), lambda b: (b, 0, 0)),
            pl.BlockSpec((1, 1, ZDIM), lambda b: (b, 0, 0)),
        ],
        out_specs=[scal] * 5,
        out_shape=out_shape,
    )(recon_xyz, hand_xyz, vn, obj_t, obj_pts, vw_col,
      rx3, x3, mu3, lv3)

    loss, param_loss, ho_loss, recon_loss, kld = [o.reshape(()) for o in outs]
    return (loss, param_loss, ho_loss, recon_loss, kld)


# speedup vs baseline: 1.5369x; 1.0495x over previous
import functools

import jax
import jax.numpy as jnp
from jax import lax
from jax.experimental import pallas as pl
from jax.experimental.pallas import tpu as pltpu
from jax.experimental.pallas import tpu_sc as plsc

B, V, NF, NO, PDIM, ZDIM = 16, 778, 1538, 2048, 61, 64
KL_COEF = 0.005
BIG_I = 2 ** 30
NFP = 1600
QC = 512
VP = 784
VP3 = VP * 3
NF3P = NFP * 3

_DNT = (((0,), (0,)), ((), ()))


def _sc_normals_body(vab_hbm, faces_hbm, out_hbm, verts_v, faces_v, vn_v):
    wid = lax.axis_index("s") * 2 + lax.axis_index("c")
    b = wid // 2
    m = wid % 2
    pltpu.sync_copy(vab_hbm.at[b, m], verts_v)
    pltpu.sync_copy(faces_hbm.at[b], faces_v)

    zero16 = jnp.zeros((16,), jnp.float32)

    def _zero(i, c):
        vn_v[pl.ds(i * 16, 16)] = zero16
        return c

    lax.fori_loop(0, VP3 // 16, _zero, 0)

    def _face_chunk(i, c):
        base = i * 16
        i0 = faces_v[pl.ds(base, 16)] * 3
        i1 = faces_v[pl.ds(NFP + base, 16)] * 3
        i2 = faces_v[pl.ds(2 * NFP + base, 16)] * 3
        v0x = plsc.load_gather(verts_v, [i0])
        v0y = plsc.load_gather(verts_v, [i0 + 1])
        v0z = plsc.load_gather(verts_v, [i0 + 2])
        v1x = plsc.load_gather(verts_v, [i1])
        v1y = plsc.load_gather(verts_v, [i1 + 1])
        v1z = plsc.load_gather(verts_v, [i1 + 2])
        v2x = plsc.load_gather(verts_v, [i2])
        v2y = plsc.load_gather(verts_v, [i2 + 1])
        v2z = plsc.load_gather(verts_v, [i2 + 2])
        e1x, e1y, e1z = v1x - v0x, v1y - v0y, v1z - v0z
        e2x, e2y, e2z = v2x - v0x, v2y - v0y, v2z - v0z
        fx = e1y * e2z - e1z * e2y
        fy = e1z * e2x - e1x * e2z
        fz = e1x * e2y - e1y * e2x
        for ic in (i0, i1, i2):
            plsc.addupdate_scatter(vn_v, [ic], fx)
            plsc.addupdate_scatter(vn_v, [ic + 1], fy)
            plsc.addupdate_scatter(vn_v, [ic + 2], fz)
        return c

    lax.fori_loop(0, NFP // 16, _face_chunk, 0)
    pltpu.sync_copy(vn_v, out_hbm.at[b, m])


def _sc_normals(vab_flat, faces_flat):
    mesh = plsc.VectorSubcoreMesh(core_axis_name="c", subcore_axis_name="s")
    fn = functools.partial(
        pl.kernel,
        mesh=mesh,
        compiler_params=pltpu.CompilerParams(needs_layout_passes=False),
        out_type=jax.ShapeDtypeStruct((B, 2, VP3), jnp.float32),
        scratch_types=[
            pltpu.VMEM((VP3,), jnp.float32),
            pltpu.VMEM((NF3P,), jnp.int32),
            pltpu.VMEM((VP3,), jnp.float32),
        ],
    )(_sc_normals_body)
    return fn(vab_flat, faces_flat)


def _loss_kernel(va_ref, vb_ref, vn_ref, objt_ref, objr_ref, vw_ref,
                 rx_ref, xx_ref, mu_ref, lv_ref,
                 loss_ref, param_ref, ho_ref, recon_ref, kld_ref):
    b = pl.program_id(0)

    @pl.when(b == 0)
    def _init():
        z = jnp.zeros((1, 1), jnp.float32)
        loss_ref[:, :] = z
        param_ref[:, :] = z
        ho_ref[:, :] = z
        recon_ref[:, :] = z
        kld_ref[:, :] = z

    va = va_ref[0]
    vb = vb_ref[0]
    vw = vw_ref[:]
    rx = rx_ref[0]
    xx = xx_ref[0]
    mu = mu_ref[0]
    lv = lv_ref[0]

    def _unit(vn):
        n = jnp.sqrt(jnp.sum(vn * vn, axis=1, keepdims=True))
        return vn / jnp.maximum(n, 1e-6)

    wa = jnp.concatenate([va, _unit(vn_ref[0, 0])], axis=1)
    wb = jnp.concatenate([vb, _unit(vn_ref[0, 1])], axis=1)

    h2a = jnp.sum(va * va, axis=1, keepdims=True)
    h2b = jnp.sum(vb * vb, axis=1, keepdims=True)
    iota_p = jax.lax.broadcasted_iota(jnp.int32, (V, QC), 0)
    iota_q = jax.lax.broadcasted_iota(jnp.int32, (V, QC), 1)

    imax = jnp.iinfo(jnp.int32).max
    st_a = [jnp.full((V, 1), imax, jnp.int32), jnp.zeros((V, 3), jnp.float32)]
    st_b = [jnp.full((V, 1), imax, jnp.int32), jnp.zeros((V, 3), jnp.float32)]
    hi_mask = jnp.int32(-1024)
    ldo = 0.0
    for k in range(NO // QC):
        objc = objt_ref[0, :, k * QC:(k + 1) * QC]
        objr = objr_ref[0, k * QC:(k + 1) * QC, :]
        o2 = jnp.sum(objc * objc, axis=0, keepdims=True)

        def _signed(verts, h2, w6, st):
            d = jnp.maximum(h2 + o2 - 2.0 * jnp.dot(verts, objc), 0.0)
            dq = jax.lax.bitcast_convert_type(d, jnp.int32) & hi_mask
            ckey = dq | iota_p
            ckmin = jnp.min(ckey, axis=0, keepdims=True)
            cmask = (ckey == ckmin).astype(jnp.float32)
            sel = jax.lax.dot_general(cmask, w6, _DNT)
            dx = objr[:, 0:1] - sel[:, 0:1]
            dy = objr[:, 1:2] - sel[:, 1:2]
            dz = objr[:, 2:3] - sel[:, 2:3]
            mag = jnp.sqrt(dx * dx + dy * dy + dz * dz)
            dotn = sel[:, 3:4] * dx + sel[:, 4:5] * dy + sel[:, 5:6] * dz
            sgn = jnp.where(dotn > 0.0, 1.0,
                            jnp.where(dotn < 0.0, -1.0, 0.0))
            rkey = dq | iota_q
            rkmin = jnp.min(rkey, axis=1, keepdims=True)
            rmask = (rkey == rkmin).astype(jnp.float32)
            rsel = jnp.dot(rmask, objr)
            upd = rkmin < st[0]
            st[0] = jnp.where(upd, rkmin, st[0])
            st[1] = jnp.where(upd, rsel, st[1])
            return mag * sgn

        o2h_a = _signed(va, h2a, wa, st_a)
        o2h_b = _signed(vb, h2b, wb, st_b)

        w_dist = (o2h_b < 0.01) & (o2h_b > -0.005)
        w = jnp.where(w_dist, 1.0, 0.1)
        w = jnp.where(o2h_a < 0.0, 1.5, w)
        ldo = ldo + jnp.sum(jnp.abs(o2h_a - o2h_b) * w)

    def _rownorm(verts, st):
        e = verts - st[1]
        return jnp.sqrt(jnp.sum(e * e, axis=1, keepdims=True))

    h2o_a = _rownorm(va, st_a)
    h2o_b = _rownorm(vb, st_b)
    w2 = jnp.exp(0.4 * jnp.log(vw))
    ldh = jnp.sum(jnp.abs(jnp.abs(h2o_a) - jnp.abs(h2o_b)) * w2)

    scale = 1.0 - KL_COEF
    ho_p = (35.0 * scale / (B * V)) * ldh + (30.0 * scale / (B * NO)) * ldo

    dpx = rx - xx
    param_p = jnp.sum(dpx * dpx) / B
    dv = va - vb
    recon_p = jnp.sum(dv * dv) / B
    kld_p = -0.5 * jnp.sum(1.0 + lv - mu * mu - jnp.exp(lv)) / B

    def _acc(ref, val):
        ref[:, :] = ref[:, :] + jnp.full((1, 1), 1.0, jnp.float32) * val

    _acc(loss_ref, (recon_p + kld_p) + 0.1 * param_p + 10.0 * ho_p)
    _acc(param_ref, param_p)
    _acc(ho_ref, ho_p)
    _acc(recon_ref, recon_p)
    _acc(kld_ref, kld_p)


def kernel(recon_x, x, mu, logvar, recon_xyz, hand_xyz, hand_faces, obj_pts,
           v_weights):
    vab = jnp.stack([recon_xyz, hand_xyz], axis=1)
    vab_flat = jnp.pad(vab, ((0, 0), (0, 0), (0, VP - V), (0, 0))
                       ).reshape(B, 2, VP3)
    faces_pad = jnp.pad(hand_faces, ((0, 0), (0, NFP - NF), (0, 0)),
                        constant_values=V)
    faces_flat = jnp.swapaxes(faces_pad, 1, 2).reshape(B, NF3P)
    vn = _sc_normals(vab_flat, faces_flat)
    vn = vn.reshape(B, 2, VP, 3)[:, :, :V, :]

    obj_t = jnp.swapaxes(obj_pts, 1, 2)
    vw_col = v_weights.reshape(V, 1)
    rx3 = recon_x.reshape(B, 1, PDIM)
    x3 = x.reshape(B, 1, PDIM)
    mu3 = mu.reshape(B, 1, ZDIM)
    lv3 = logvar.reshape(B, 1, ZDIM)

    out_shape = [jax.ShapeDtypeStruct((1, 1), jnp.float32)] * 5
    scal = pl.BlockSpec((1, 1), lambda b: (0, 0))
    outs = pl.pallas_call(
        _loss_kernel,
        grid=(B,),
        in_specs=[
            pl.BlockSpec((1, V, 3), lambda b: (b, 0, 0)),
            pl.BlockSpec((1, V, 3), lambda b: (b, 0, 0)),
            pl.BlockSpec((1, 2, V, 3), lambda b: (b, 0, 0, 0)),
            pl.BlockSpec((1, 3, NO), lambda b: (b, 0, 0)),
            pl.BlockSpec((1, NO, 3), lambda b: (b, 0, 0)),
            pl.BlockSpec((V, 1), lambda b: (0, 0)),
            pl.BlockSpec((1, 1, PDIM), lambda b: (b, 0, 0)),
            pl.BlockSpec((1, 1, PDIM), lambda b: (b, 0, 0)),
            pl.BlockSpec((1, 1, ZDIM), lambda b: (b, 0, 0)),
            pl.BlockSpec((1, 1, ZDIM), lambda b: (b, 0, 0)),
        ],
        out_specs=[scal] * 5,
        out_shape=out_shape,
    )(recon_xyz, hand_xyz, vn, obj_t, obj_pts, vw_col,
      rx3, x3, mu3, lv3)

    loss, param_loss, ho_loss, recon_loss, kld = [o.reshape(()) for o in outs]
    return (loss, param_loss, ho_loss, recon_loss, kld)


# speedup vs baseline: 1.6418x; 1.0683x over previous
import functools

import jax
import jax.numpy as jnp
from jax import lax
from jax.experimental import pallas as pl
from jax.experimental.pallas import tpu as pltpu
from jax.experimental.pallas import tpu_sc as plsc

B, V, NF, NO, PDIM, ZDIM = 16, 778, 1538, 2048, 61, 64
KL_COEF = 0.005
BIG_I = 2 ** 30
NFP = 1600
QC = 512
VP = 784
VP3 = VP * 3
NF3P = NFP * 3

_DNT = (((0,), (0,)), ((), ()))


def _sc_normals_body(vab_hbm, faces_hbm, out_hbm, verts_v, faces_v, vn_v):
    wid = lax.axis_index("s") * 2 + lax.axis_index("c")
    b = wid // 2
    m = wid % 2
    pltpu.sync_copy(vab_hbm.at[b, m], verts_v)
    pltpu.sync_copy(faces_hbm.at[b], faces_v)

    zero16 = jnp.zeros((16,), jnp.float32)

    def _zero(i, c):
        vn_v[pl.ds(i * 16, 16)] = zero16
        return c

    lax.fori_loop(0, VP3 // 16, _zero, 0)

    def _face_chunk(i, c):
        base = i * 16
        i0 = faces_v[pl.ds(base, 16)] * 3
        i1 = faces_v[pl.ds(NFP + base, 16)] * 3
        i2 = faces_v[pl.ds(2 * NFP + base, 16)] * 3
        v0x = plsc.load_gather(verts_v, [i0])
        v0y = plsc.load_gather(verts_v, [i0 + 1])
        v0z = plsc.load_gather(verts_v, [i0 + 2])
        v1x = plsc.load_gather(verts_v, [i1])
        v1y = plsc.load_gather(verts_v, [i1 + 1])
        v1z = plsc.load_gather(verts_v, [i1 + 2])
        v2x = plsc.load_gather(verts_v, [i2])
        v2y = plsc.load_gather(verts_v, [i2 + 1])
        v2z = plsc.load_gather(verts_v, [i2 + 2])
        e1x, e1y, e1z = v1x - v0x, v1y - v0y, v1z - v0z
        e2x, e2y, e2z = v2x - v0x, v2y - v0y, v2z - v0z
        fx = e1y * e2z - e1z * e2y
        fy = e1z * e2x - e1x * e2z
        fz = e1x * e2y - e1y * e2x
        for ic in (i0, i1, i2):
            plsc.addupdate_scatter(vn_v, [ic], fx)
            plsc.addupdate_scatter(vn_v, [ic + 1], fy)
            plsc.addupdate_scatter(vn_v, [ic + 2], fz)
        return c

    lax.fori_loop(0, NFP // 16, _face_chunk, 0)
    pltpu.sync_copy(vn_v, out_hbm.at[b, m])


def _sc_normals(vab_flat, faces_flat):
    mesh = plsc.VectorSubcoreMesh(core_axis_name="c", subcore_axis_name="s")
    fn = functools.partial(
        pl.kernel,
        mesh=mesh,
        compiler_params=pltpu.CompilerParams(needs_layout_passes=False),
        out_type=jax.ShapeDtypeStruct((B, 2, VP3), jnp.float32),
        scratch_types=[
            pltpu.VMEM((VP3,), jnp.float32),
            pltpu.VMEM((NF3P,), jnp.int32),
            pltpu.VMEM((VP3,), jnp.float32),
        ],
    )(_sc_normals_body)
    return fn(vab_flat, faces_flat)


def _loss_kernel(va_ref, vb_ref, vn_ref, objt_ref, objr_ref, vw_ref,
                 rx_ref, xx_ref, mu_ref, lv_ref,
                 loss_ref, param_ref, ho_ref, recon_ref, kld_ref):
    b = pl.program_id(0)

    @pl.when(b == 0)
    def _init():
        z = jnp.zeros((1, 1), jnp.float32)
        loss_ref[:, :] = z
        param_ref[:, :] = z
        ho_ref[:, :] = z
        recon_ref[:, :] = z
        kld_ref[:, :] = z

    va = va_ref[0]
    vb = vb_ref[0]
    vw = vw_ref[:]
    rx = rx_ref[0]
    xx = xx_ref[0]
    mu = mu_ref[0]
    lv = lv_ref[0]

    def _unit(vn):
        n = jnp.sqrt(jnp.sum(vn * vn, axis=1, keepdims=True))
        return vn / jnp.maximum(n, 1e-6)

    wa = jnp.concatenate([va, _unit(vn_ref[0, 0])], axis=1)
    wb = jnp.concatenate([vb, _unit(vn_ref[0, 1])], axis=1)

    def _aug(verts):
        h2 = jnp.sum(verts * verts, axis=1, keepdims=True)
        one = jnp.ones((V, 1), jnp.float32)
        return jnp.concatenate([-2.0 * verts, h2, one], axis=1)

    ava = _aug(va)
    avb = _aug(vb)
    iota_p = jax.lax.broadcasted_iota(jnp.int32, (V, QC), 0)
    iota_q = jax.lax.broadcasted_iota(jnp.int32, (V, QC), 1)

    imax = jnp.iinfo(jnp.int32).max
    st_a = [jnp.full((V, 1), imax, jnp.int32), jnp.zeros((V, 3), jnp.float32)]
    st_b = [jnp.full((V, 1), imax, jnp.int32), jnp.zeros((V, 3), jnp.float32)]
    hi_mask = jnp.int32(-1024)
    ldo = 0.0
    for k in range(NO // QC):
        objc = objt_ref[0, :, k * QC:(k + 1) * QC]
        objr = objr_ref[0, k * QC:(k + 1) * QC, :]

        def _signed(averts, w6, st):
            d = jnp.maximum(jnp.dot(averts, objc), 0.0)
            dq = jax.lax.bitcast_convert_type(d, jnp.int32) & hi_mask
            ckey = dq | iota_p
            ckmin = jnp.min(ckey, axis=0, keepdims=True)
            cmask = (ckey == ckmin).astype(jnp.float32)
            sel = jax.lax.dot_general(cmask, w6, _DNT)
            dx = objr[:, 0:1] - sel[:, 0:1]
            dy = objr[:, 1:2] - sel[:, 1:2]
            dz = objr[:, 2:3] - sel[:, 2:3]
            mag = jnp.sqrt(dx * dx + dy * dy + dz * dz)
            dotn = sel[:, 3:4] * dx + sel[:, 4:5] * dy + sel[:, 5:6] * dz
            sgn = jnp.where(dotn > 0.0, 1.0,
                            jnp.where(dotn < 0.0, -1.0, 0.0))
            rkey = dq | iota_q
            rkmin = jnp.min(rkey, axis=1, keepdims=True)
            rmask = (rkey == rkmin).astype(jnp.float32)
            rsel = jnp.dot(rmask, objr)
            upd = rkmin < st[0]
            st[0] = jnp.where(upd, rkmin, st[0])
            st[1] = jnp.where(upd, rsel, st[1])
            return mag * sgn

        o2h_a = _signed(ava, wa, st_a)
        o2h_b = _signed(avb, wb, st_b)

        w_dist = (o2h_b < 0.01) & (o2h_b > -0.005)
        w = jnp.where(w_dist, 1.0, 0.1)
        w = jnp.where(o2h_a < 0.0, 1.5, w)
        ldo = ldo + jnp.sum(jnp.abs(o2h_a - o2h_b) * w)

    def _rownorm(verts, st):
        e = verts - st[1]
        return jnp.sqrt(jnp.sum(e * e, axis=1, keepdims=True))

    h2o_a = _rownorm(va, st_a)
    h2o_b = _rownorm(vb, st_b)
    w2 = jnp.exp(0.4 * jnp.log(vw))
    ldh = jnp.sum(jnp.abs(jnp.abs(h2o_a) - jnp.abs(h2o_b)) * w2)

    scale = 1.0 - KL_COEF
    ho_p = (35.0 * scale / (B * V)) * ldh + (30.0 * scale / (B * NO)) * ldo

    dpx = rx - xx
    param_p = jnp.sum(dpx * dpx) / B
    dv = va - vb
    recon_p = jnp.sum(dv * dv) / B
    kld_p = -0.5 * jnp.sum(1.0 + lv - mu * mu - jnp.exp(lv)) / B

    def _acc(ref, val):
        ref[:, :] = ref[:, :] + jnp.full((1, 1), 1.0, jnp.float32) * val

    _acc(loss_ref, (recon_p + kld_p) + 0.1 * param_p + 10.0 * ho_p)
    _acc(param_ref, param_p)
    _acc(ho_ref, ho_p)
    _acc(recon_ref, recon_p)
    _acc(kld_ref, kld_p)


def kernel(recon_x, x, mu, logvar, recon_xyz, hand_xyz, hand_faces, obj_pts,
           v_weights):
    vab = jnp.stack([recon_xyz, hand_xyz], axis=1)
    vab_flat = jnp.pad(vab, ((0, 0), (0, 0), (0, VP - V), (0, 0))
                       ).reshape(B, 2, VP3)
    faces_pad = jnp.pad(hand_faces, ((0, 0), (0, NFP - NF), (0, 0)),
                        constant_values=V)
    faces_flat = jnp.swapaxes(faces_pad, 1, 2).reshape(B, NF3P)
    vn = _sc_normals(vab_flat, faces_flat)
    vn = vn.reshape(B, 2, VP, 3)[:, :, :V, :]

    obj_t = jnp.swapaxes(obj_pts, 1, 2)
    o2_full = jnp.sum(obj_t * obj_t, axis=1, keepdims=True)
    obj_aug = jnp.concatenate(
        [obj_t, jnp.ones((B, 1, NO), jnp.float32), o2_full], axis=1)
    vw_col = v_weights.reshape(V, 1)
    rx3 = recon_x.reshape(B, 1, PDIM)
    x3 = x.reshape(B, 1, PDIM)
    mu3 = mu.reshape(B, 1, ZDIM)
    lv3 = logvar.reshape(B, 1, ZDIM)

    out_shape = [jax.ShapeDtypeStruct((1, 1), jnp.float32)] * 5
    scal = pl.BlockSpec((1, 1), lambda b: (0, 0))
    outs = pl.pallas_call(
        _loss_kernel,
        grid=(B,),
        in_specs=[
            pl.BlockSpec((1, V, 3), lambda b: (b, 0, 0)),
            pl.BlockSpec((1, V, 3), lambda b: (b, 0, 0)),
            pl.BlockSpec((1, 2, V, 3), lambda b: (b, 0, 0, 0)),
            pl.BlockSpec((1, 5, NO), lambda b: (b, 0, 0)),
            pl.BlockSpec((1, NO, 3), lambda b: (b, 0, 0)),
            pl.BlockSpec((V, 1), lambda b: (0, 0)),
            pl.BlockSpec((1, 1, PDIM), lambda b: (b, 0, 0)),
            pl.BlockSpec((1, 1, PDIM), lambda b: (b, 0, 0)),
            pl.BlockSpec((1, 1, ZDIM), lambda b: (b, 0, 0)),
            pl.BlockSpec((1, 1, ZDIM), lambda b: (b, 0, 0)),
        ],
        out_specs=[scal] * 5,
        out_shape=out_shape,
    )(recon_xyz, hand_xyz, vn, obj_aug, obj_pts, vw_col,
      rx3, x3, mu3, lv3)

    loss, param_loss, ho_loss, recon_loss, kld = [o.reshape(()) for o in outs]
    return (loss, param_loss, ho_loss, recon_loss, kld)
